# baseline (speedup 1.0000x reference)
import dataclasses
import functools

import jax
import jax.numpy as jnp
from jax import lax
from jax.experimental import pallas as pl
from jax.experimental.pallas import tpu as pltpu
from jax.experimental.pallas import tpu_sc as plsc

N_SPECIES = 8192
N_REACT = 65536
BATCH = 256

NC = 2
NS = 16
LANES = 16
BC = BATCH // NC
W = 64
RPT = N_REACT // NS
NCHUNK = RPT // W
BLK = 4
NSLOT = 4
STEP = NSLOT * BLK
ROWS_PER_TILE = N_SPECIES // NS

G_I1R, G_I2R01, G_R1, G_R2 = 0, W, 3 * W, 4 * W
GREC = 5 * W
GBLK = BLK * GREC


def _sc_kinetics(y2, gmeta, p1, p2):
    mesh = plsc.VectorSubcoreMesh(core_axis_name="c", subcore_axis_name="s")
    cp = pltpu.CompilerParams()
    if "needs_layout_passes" in pltpu.CompilerParams.__dataclass_fields__:
        cp = dataclasses.replace(cp, needs_layout_passes=False)

    @functools.partial(
        pl.kernel,
        out_type=jax.ShapeDtypeStruct((NC * N_SPECIES, BC), jnp.float32),
        mesh=mesh,
        compiler_params=cp,
        scratch_types=[
            pltpu.VMEM((NSLOT * GBLK,), jnp.int32),
        ] + [pltpu.VMEM((BLK, W), jnp.int32)] * NSLOT
          + [pltpu.VMEM((BLK, W), jnp.int32)] * NSLOT + [
            pltpu.VMEM((W, BC), jnp.float32),
            pltpu.VMEM((W, BC), jnp.float32),
            pltpu.VMEM((2 * W, BC), jnp.float32),
            pltpu.VMEM((2 * W, BC), jnp.float32),
            pltpu.VMEM_SHARED((N_SPECIES, BC), jnp.float32),
        ] + [pltpu.SemaphoreType.DMA] * 12,
    )
    def k(y2_hbm, gmeta_hbm, p1_hbm, p2_hbm, out_hbm,
          gm, p1_0, p1_1, p1_2, p1_3, p2_0, p2_1, p2_2, p2_3,
          f0, f1, ab0, ab1, acc,
          fg0, fg1, ag0, ag1, s10, s11, s20, s21, m0, m1, m2, m3):
        c = lax.axis_index("c")
        s = lax.axis_index("s")
        yoff = c * N_SPECIES
        f = (f0, f1)
        ab = (ab0, ab1)
        p1x = (p1_0, p1_1, p1_2, p1_3)
        p2x = (p2_0, p2_1, p2_2, p2_3)
        sem_fg = (fg0, fg1)
        sem_ag = (ag0, ag1)
        sem_s1 = (s10, s11)
        sem_s2 = (s20, s21)
        sem_m = (m0, m1, m2, m3)

        def meta_copies(kc0, sb):
            row = s * NCHUNK + kc0
            return (
                pltpu.make_async_copy(gmeta_hbm.at[pl.ds(row * GREC, GBLK)],
                                      gm.at[pl.ds(sb * GBLK, GBLK)], sem_m[sb]),
                pltpu.make_async_copy(p1_hbm.at[pl.ds(row, BLK)], p1x[sb],
                                      sem_m[sb]),
                pltpu.make_async_copy(p2_hbm.at[pl.ds(row, BLK)], p2x[sb],
                                      sem_m[sb]),
            )

        def start_meta(kc0, sb):
            for cp_ in meta_copies(kc0, sb):
                cp_.start()

        def wait_meta(sb):
            for cp_ in meta_copies(0, sb):
                cp_.wait()

        def offset_block(sb):
            for ci in range(BLK):
                base = sb * GBLK + ci * GREC
                for g in range(3 * W // LANES):
                    sl = pl.ds(base + g * LANES, LANES)
                    gm[sl] = gm[sl] + yoff

        def rate16(sb, ci, roff, w):
            base = sb * GBLK + ci * GREC + roff
            bits = plsc.load_gather(
                gm, [jnp.full((LANES,), base, jnp.int32) + w])
            return plsc.bitcast(bits, jnp.float32)

        def first_gather(bj, sb, ci):
            base = sb * GBLK + ci * GREC + G_I1R
            return pltpu.make_async_copy(
                y2_hbm.at[gm.at[pl.ds(base, W)]], f[bj], sem_fg[bj])

        def second_gather(bj, sb, ci):
            base = sb * GBLK + ci * GREC + G_I2R01
            return pltpu.make_async_copy(
                y2_hbm.at[gm.at[pl.ds(base, 2 * W)]], ab[bj], sem_ag[bj])

        class _Scatter:
            def __init__(self, src, dst, sem):
                self.src, self.dst, self.sem = src, dst, sem

            def start(self):
                pltpu.async_copy(self.src, self.dst, self.sem, add=True)

            def wait(self):
                pltpu.make_async_copy(self.src, self.dst, self.sem).wait()

        def first_scatter(bj, sb, ci):
            return _Scatter(f[bj], acc.at[p1x[sb].at[ci]], sem_s1[bj])

        def second_scatter(bj, sb, ci):
            return _Scatter(ab[bj].at[pl.ds(0, W)], acc.at[p2x[sb].at[ci]],
                            sem_s2[bj])

        def first_multiply(bj, sb, ci):
            @plsc.parallel_loop(0, W, 1, unroll=8)
            def _(w):
                r16 = rate16(sb, ci, G_R1, w)
                for g in range(BC // LANES):
                    sl = pl.ds(g * LANES, LANES)
                    f[bj][w, sl] = f[bj][w, sl] * r16

        def second_multiply(bj, sb, ci):
            @plsc.parallel_loop(0, W, 1, unroll=8)
            def _(w):
                r16 = rate16(sb, ci, G_R2, w)
                for g in range(BC // LANES):
                    sl = pl.ds(g * LANES, LANES)
                    ab[bj][w, sl] = ab[bj][w, sl] * ab[bj][W + w, sl] * r16

        for sb in range(NSLOT):
            start_meta(sb * BLK, sb)
        wait_meta(0)
        offset_block(0)
        for t in range(2):
            second_gather(t, 0, t).start()

        @pl.loop(0, W)
        def _(w):
            for g in range(BC // LANES):
                f0[w, pl.ds(g * LANES, LANES)] = jnp.zeros((LANES,), jnp.float32)

        @pl.loop(0, ROWS_PER_TILE // W)
        def _(blk):
            pltpu.sync_copy(f0, acc.at[pl.ds(s * ROWS_PER_TILE + blk * W, W)])

        for t in range(2):
            first_gather(t, 0, t).start()

        plsc.subcore_barrier()

        @pl.loop(0, NCHUNK, step=STEP)
        def _(k0):
            for j in range(STEP):
                bj = j % 2
                nb = 1 - bj
                sb, ci = j // BLK, j % BLK
                nsb, nci = ((j + 1) // BLK) % NSLOT, (j + 1) % BLK
                nnsb, nnci = ((j + 2) // BLK) % NSLOT, (j + 2) % BLK

                if nnci == 0:
                    if j == STEP - 2:
                        @pl.when(k0 < NCHUNK - STEP)
                        def _():
                            wait_meta(nnsb)
                            offset_block(nnsb)
                    else:
                        wait_meta(nnsb)
                        offset_block(nnsb)

                first_gather(bj, sb, ci).wait()
                first_multiply(bj, sb, ci)
                first_scatter(bj, sb, ci).start()

                second_gather(bj, sb, ci).wait()
                second_multiply(bj, sb, ci)
                second_scatter(bj, sb, ci).start()

                first_scatter(bj, sb, ci).wait()
                second_scatter(bj, sb, ci).wait()

                def gathers_ahead():
                    first_gather(bj, nnsb, nnci).start()
                    second_gather(bj, nnsb, nnci).start()

                if j < STEP - 2:
                    gathers_ahead()
                else:
                    @pl.when(k0 < NCHUNK - STEP)
                    def _():
                        gathers_ahead()

                if j % BLK == 1:
                    nxt = (j // BLK + NSLOT - 1) % NSLOT
                    first_new = 3 * BLK + j - 1
                    if j == 1:
                        @pl.when((k0 > 0) & (k0 < NCHUNK - first_new))
                        def _():
                            start_meta(k0 + first_new, nxt)
                    else:
                        @pl.when(k0 < NCHUNK - first_new)
                        def _():
                            start_meta(k0 + first_new, nxt)

        plsc.subcore_barrier()

        def drain(blk):
            row = s * ROWS_PER_TILE + blk * 2 * W
            return pltpu.make_async_copy(acc.at[pl.ds(row, 2 * W)],
                                         out_hbm.at[pl.ds(yoff + row, 2 * W)],
                                         sem_m[blk])

        for blk in range(ROWS_PER_TILE // (2 * W)):
            drain(blk).start()
        for blk in range(ROWS_PER_TILE // (2 * W)):
            drain(blk).wait()

    return k(y2, gmeta, p1, p2)


def kernel(t_in, y_in, inds_1r, inds_1p, rate_1, inds_2r0, inds_2r1, inds_2p, rate_2):
    del t_in
    y2 = y_in.reshape(NC, BC, N_SPECIES).transpose(0, 2, 1).reshape(NC * N_SPECIES, BC)
    chunked = lambda v: v.astype(jnp.int32).reshape(N_REACT // W, W)
    fbits = lambda v: lax.bitcast_convert_type(v, jnp.int32).reshape(N_REACT // W, W)
    gmeta = jnp.concatenate([
        chunked(inds_1r), chunked(inds_2r0), chunked(inds_2r1),
        fbits(rate_1), fbits(rate_2),
    ], axis=1).reshape(-1)
    out2 = _sc_kinetics(y2, gmeta, chunked(inds_1p), chunked(inds_2p))
    return out2.reshape(NC, N_SPECIES, BC).transpose(0, 2, 1).reshape(BATCH, N_SPECIES)


# speedup vs baseline: 1.0372x; 1.0372x over previous
import dataclasses
import functools

import jax
import jax.numpy as jnp
from jax import lax
from jax.experimental import pallas as pl
from jax.experimental.pallas import tpu as pltpu
from jax.experimental.pallas import tpu_sc as plsc

N_SPECIES = 8192
N_REACT = 65536
BATCH = 256

NC = 2
NS = 16
LANES = 16
BC = BATCH // NC
W = 64
RPT = N_REACT // NS
NCHUNK = RPT // W
BLK = 4
NSLOT = 4
STEP = NSLOT * BLK
ROWS_PER_TILE = N_SPECIES // NS

G_I1R, G_I2R01, G_R1, G_R2 = 0, W, 3 * W, 4 * W
GREC = 5 * W
GBLK = BLK * GREC


def _sc_kinetics(y2, gmeta, p1, p2):
    mesh = plsc.VectorSubcoreMesh(core_axis_name="c", subcore_axis_name="s")
    cp = pltpu.CompilerParams()
    if "needs_layout_passes" in pltpu.CompilerParams.__dataclass_fields__:
        cp = dataclasses.replace(cp, needs_layout_passes=False)

    @functools.partial(
        pl.kernel,
        out_type=jax.ShapeDtypeStruct((NC * N_SPECIES, BC), jnp.float32),
        mesh=mesh,
        compiler_params=cp,
        scratch_types=[
            pltpu.VMEM((NSLOT * GBLK,), jnp.int32),
        ] + [pltpu.VMEM((BLK, W), jnp.int32)] * NSLOT
          + [pltpu.VMEM((BLK, W), jnp.int32)] * NSLOT + [
            pltpu.VMEM((W, BC), jnp.float32),
            pltpu.VMEM((W, BC), jnp.float32),
            pltpu.VMEM((2 * W, BC), jnp.float32),
            pltpu.VMEM((2 * W, BC), jnp.float32),
            pltpu.VMEM_SHARED((N_SPECIES, BC), jnp.float32),
        ] + [pltpu.SemaphoreType.DMA] * 12,
    )
    def k(y2_hbm, gmeta_hbm, p1_hbm, p2_hbm, out_hbm,
          gm, p1_0, p1_1, p1_2, p1_3, p2_0, p2_1, p2_2, p2_3,
          f0, f1, ab0, ab1, acc,
          fg0, fg1, ag0, ag1, s10, s11, s20, s21, m0, m1, m2, m3):
        c = lax.axis_index("c")
        s = lax.axis_index("s")
        yoff = c * N_SPECIES
        f = (f0, f1)
        ab = (ab0, ab1)
        p1x = (p1_0, p1_1, p1_2, p1_3)
        p2x = (p2_0, p2_1, p2_2, p2_3)
        sem_fg = (fg0, fg1)
        sem_ag = (ag0, ag1)
        sem_s1 = (s10, s11)
        sem_s2 = (s20, s21)
        sem_m = (m0, m1, m2, m3)

        def meta_copies(kc0, sb):
            row = s * NCHUNK + kc0
            return (
                pltpu.make_async_copy(gmeta_hbm.at[pl.ds(row * GREC, GBLK)],
                                      gm.at[pl.ds(sb * GBLK, GBLK)], sem_m[sb]),
                pltpu.make_async_copy(p1_hbm.at[pl.ds(row, BLK)], p1x[sb],
                                      sem_m[sb]),
                pltpu.make_async_copy(p2_hbm.at[pl.ds(row, BLK)], p2x[sb],
                                      sem_m[sb]),
            )

        def start_meta(kc0, sb):
            for cp_ in meta_copies(kc0, sb):
                cp_.start()

        def wait_meta(sb):
            for cp_ in meta_copies(0, sb):
                cp_.wait()

        def offset_block(sb):
            for ci in range(BLK):
                base = sb * GBLK + ci * GREC
                for g in range(3 * W // LANES):
                    sl = pl.ds(base + g * LANES, LANES)
                    gm[sl] = gm[sl] + yoff

        def rate16(sb, ci, roff, w):
            base = sb * GBLK + ci * GREC + roff
            bits = plsc.load_gather(
                gm, [jnp.full((LANES,), base, jnp.int32) + w])
            return plsc.bitcast(bits, jnp.float32)

        def first_gather(bj, sb, ci):
            base = sb * GBLK + ci * GREC + G_I1R
            return pltpu.make_async_copy(
                y2_hbm.at[gm.at[pl.ds(base, W)]], f[bj], sem_fg[bj])

        def second_gather(bj, sb, ci):
            base = sb * GBLK + ci * GREC + G_I2R01
            return pltpu.make_async_copy(
                y2_hbm.at[gm.at[pl.ds(base, 2 * W)]], ab[bj], sem_ag[bj])

        class _Scatter:
            def __init__(self, src, dst, sem):
                self.src, self.dst, self.sem = src, dst, sem

            def start(self):
                pltpu.async_copy(self.src, self.dst, self.sem, add=True)

            def wait(self):
                pltpu.make_async_copy(self.src, self.dst, self.sem).wait()

        def first_scatter(bj, sb, ci):
            return _Scatter(f[bj], acc.at[p1x[sb].at[ci]], sem_s1[bj])

        def second_scatter(bj, sb, ci):
            return _Scatter(ab[bj].at[pl.ds(0, W)], acc.at[p2x[sb].at[ci]],
                            sem_s2[bj])

        def first_multiply(bj, sb, ci):
            @plsc.parallel_loop(0, W, 1, unroll=4)
            def _(w):
                r16 = rate16(sb, ci, G_R1, w)
                for g in range(BC // LANES):
                    sl = pl.ds(g * LANES, LANES)
                    f[bj][w, sl] = f[bj][w, sl] * r16

        def second_multiply(bj, sb, ci):
            @plsc.parallel_loop(0, W, 1, unroll=4)
            def _(w):
                r16 = rate16(sb, ci, G_R2, w)
                for g in range(BC // LANES):
                    sl = pl.ds(g * LANES, LANES)
                    ab[bj][w, sl] = ab[bj][w, sl] * ab[bj][W + w, sl] * r16

        for sb in range(NSLOT):
            start_meta(sb * BLK, sb)
        wait_meta(0)
        offset_block(0)
        for t in range(2):
            second_gather(t, 0, t).start()

        @pl.loop(0, W)
        def _(w):
            for g in range(BC // LANES):
                f0[w, pl.ds(g * LANES, LANES)] = jnp.zeros((LANES,), jnp.float32)

        @pl.loop(0, ROWS_PER_TILE // W)
        def _(blk):
            pltpu.sync_copy(f0, acc.at[pl.ds(s * ROWS_PER_TILE + blk * W, W)])

        for t in range(2):
            first_gather(t, 0, t).start()

        plsc.subcore_barrier()

        @pl.loop(0, NCHUNK, step=STEP)
        def _(k0):
            for j in range(STEP):
                bj = j % 2
                nb = 1 - bj
                sb, ci = j // BLK, j % BLK
                nsb, nci = ((j + 1) // BLK) % NSLOT, (j + 1) % BLK
                nnsb, nnci = ((j + 2) // BLK) % NSLOT, (j + 2) % BLK

                if nnci == 0:
                    if j == STEP - 2:
                        @pl.when(k0 < NCHUNK - STEP)
                        def _():
                            wait_meta(nnsb)
                            offset_block(nnsb)
                    else:
                        wait_meta(nnsb)
                        offset_block(nnsb)

                first_gather(bj, sb, ci).wait()
                first_multiply(bj, sb, ci)
                first_scatter(bj, sb, ci).start()

                second_gather(bj, sb, ci).wait()
                second_multiply(bj, sb, ci)
                second_scatter(bj, sb, ci).start()

                first_scatter(bj, sb, ci).wait()
                second_scatter(bj, sb, ci).wait()

                def gathers_ahead():
                    first_gather(bj, nnsb, nnci).start()
                    second_gather(bj, nnsb, nnci).start()

                if j < STEP - 2:
                    gathers_ahead()
                else:
                    @pl.when(k0 < NCHUNK - STEP)
                    def _():
                        gathers_ahead()

                if j % BLK == 1:
                    nxt = (j // BLK + NSLOT - 1) % NSLOT
                    first_new = 3 * BLK + j - 1
                    if j == 1:
                        @pl.when((k0 > 0) & (k0 < NCHUNK - first_new))
                        def _():
                            start_meta(k0 + first_new, nxt)
                    else:
                        @pl.when(k0 < NCHUNK - first_new)
                        def _():
                            start_meta(k0 + first_new, nxt)

        plsc.subcore_barrier()

        def drain(blk):
            row = s * ROWS_PER_TILE + blk * 2 * W
            return pltpu.make_async_copy(acc.at[pl.ds(row, 2 * W)],
                                         out_hbm.at[pl.ds(yoff + row, 2 * W)],
                                         sem_m[blk])

        for blk in range(ROWS_PER_TILE // (2 * W)):
            drain(blk).start()
        for blk in range(ROWS_PER_TILE // (2 * W)):
            drain(blk).wait()

    return k(y2, gmeta, p1, p2)


def kernel(t_in, y_in, inds_1r, inds_1p, rate_1, inds_2r0, inds_2r1, inds_2p, rate_2):
    del t_in
    y2 = y_in.reshape(NC, BC, N_SPECIES).transpose(0, 2, 1).reshape(NC * N_SPECIES, BC)
    chunked = lambda v: v.astype(jnp.int32).reshape(N_REACT // W, W)
    fbits = lambda v: lax.bitcast_convert_type(v, jnp.int32).reshape(N_REACT // W, W)
    gmeta = jnp.concatenate([
        chunked(inds_1r), chunked(inds_2r0), chunked(inds_2r1),
        fbits(rate_1), fbits(rate_2),
    ], axis=1).reshape(-1)
    out2 = _sc_kinetics(y2, gmeta, chunked(inds_1p), chunked(inds_2p))
    return out2.reshape(NC, N_SPECIES, BC).transpose(0, 2, 1).reshape(BATCH, N_SPECIES)
